# Initial kernel scaffold; baseline (speedup 1.0000x reference)
import jax
import jax.numpy as jnp
from jax.experimental import pallas as pl
from jax.experimental.pallas import tpu as pltpu


def _body(x_ref, o_ref):
    v = x_ref[...]
    r = jnp.tanh(v) + jnp.log(v + 2.0) + jax.lax.rsqrt(v + 2.0)
    r = r + jnp.sin(v) + jnp.cos(v) + jnp.tan(v)
    r = r + jnp.arctan(v) + jnp.arctanh(v * 0.4) + jnp.exp(v) + jnp.log1p(v + 1.1)
    o_ref[...] = r


def kernel(x, edge_index, batch, EW1, Eb1, EW2, Eb2, GW1, Gb1, GW2, Gb2, u, tau_raw):
    y = pl.pallas_call(
        _body,
        out_shape=jax.ShapeDtypeStruct(x.shape, x.dtype),
    )(x)
    return y[:64, :64]


# speedup vs baseline: 6.8693x; 6.8693x over previous
import functools
import numpy as np
import jax
import jax.numpy as jnp
from jax import lax
from jax.experimental import pallas as pl
from jax.experimental.pallas import tpu as pltpu
from jax.experimental.pallas import tpu_sc as plsc

N = 10000
E = 320000
IN_DIM = 128
HID = 128
OUT = 64
NG = 64
GATE_HID = 32
CURVS = (-1.0, 0.0, 1.0)
TAU_MIN = 0.05
TAU_MAX = 10.0
EPS = 1e-15

F32 = jnp.float32

NC = 2
NS = 16
NW = NC * NS
EB = 128
NBATCH = -(-E // (EB * NW))
EPW = NBATCH * EB
EP = EPW * NW
ACC_ROWS = 10240
RPT = ACC_ROWS // NS
RPSUB = 160
DUMMY = N

RB = 2000
NGRID = N // RB


def _norm(v):
    return jnp.sqrt(jnp.clip(jnp.sum(v * v, axis=-1, keepdims=True), EPS, None))


def _atan_pos(z):
    t = z
    for _ in range(3):
        t = t / (1.0 + jnp.sqrt(1.0 + t * t))
    t2 = t * t
    p = t * (1.0 + t2 * (-1.0 / 3.0 + t2 * (0.2 + t2 * (-1.0 / 7.0 + t2 / 9.0))))
    return 8.0 * p


def _atanh(z):
    return 0.5 * (jnp.log1p(z) - jnp.log1p(-z))


def _tan_k(t, k):
    if k > 0:
        sk = np.sqrt(k)
        return jnp.tan(sk * t) / sk
    if k < 0:
        sk = np.sqrt(-k)
        return jnp.tanh(sk * t) / sk
    return t


def _artan_k(t, k):
    if k > 0:
        sk = np.sqrt(k)
        return _atan_pos(sk * t) / sk
    if k < 0:
        sk = np.sqrt(-k)
        return _atanh(jnp.clip(sk * t, -1.0 + 1e-7, 1.0 - 1e-7)) / sk
    return t


def _project(v, k):
    if k < 0:
        maxn = (1.0 - 1e-3) / np.sqrt(-k)
        n = _norm(v)
        return jnp.where(n > maxn, v / n * maxn, v)
    return v


def _expmap0(v, k):
    n = _norm(v)
    return _project(_tan_k(n, k) * v / n, k)


def _logmap0(v, k):
    n = _norm(v)
    return _artan_k(n, k) * v / n


def _mobius_add(x, y, k):
    x2 = jnp.sum(x * x, -1, keepdims=True)
    y2 = jnp.sum(y * y, -1, keepdims=True)
    xy = jnp.sum(x * y, -1, keepdims=True)
    num = (1.0 - 2.0 * k * xy - k * y2) * x + (1.0 + k * x2) * y
    den = 1.0 - 2.0 * k * xy + k * k * x2 * y2
    den = jnp.where(jnp.abs(den) < EPS, EPS, den)
    return num / den


def _mdist(x, y, k):
    return 2.0 * _artan_k(jnp.squeeze(_norm(_mobius_add(-x, y, k)), -1), k)


def _matmul_t(a, w):
    return lax.dot_general(a, w, (((1,), (1,)), ((), ())),
                           preferred_element_type=F32)


def _kappa_dense(o, W, b, k):
    lm = _logmap0(o, k)
    y = _matmul_t(lm, W)
    xl = _expmap0(y, k)
    kb = _expmap0(b[None, :], k)
    xl = _project(_mobius_add(xl, kb, k), k)
    return _logmap0(xl, k)


def _edge_loop(rowp, colp, table, acc, ridx, cidx, rbuf, sem, base):
    def body(b, _):
        off = pl.multiple_of(base + b * EB, EB)
        pltpu.sync_copy(rowp.at[pl.ds(off, EB)], ridx)
        pltpu.async_copy(table.at[ridx], rbuf, sem).wait()
        pltpu.sync_copy(colp.at[pl.ds(off, EB)], cidx)
        pltpu.sync_copy(rbuf, acc.at[cidx], add=True)
        return ()

    lax.fori_loop(0, NBATCH, body, (), unroll=False)


def _deg_body(colp, z1, out, cidx, ones_v, bounce, acc, sem):
    c = lax.axis_index("c")
    s = lax.axis_index("s")
    base = (c * NS + s) * EPW
    for j in range(EB // 16):
        ones_v[pl.ds(j * 16, 16)] = jnp.ones((16,), F32)
    rows = pl.ds(s * RPT, RPT)
    pltpu.sync_copy(z1.at[rows], bounce)
    pltpu.sync_copy(bounce, acc.at[rows])
    plsc.subcore_barrier()

    def body(b, _):
        off = pl.multiple_of(base + b * EB, EB)
        pltpu.sync_copy(colp.at[pl.ds(off, EB)], cidx)
        pltpu.sync_copy(ones_v, acc.at[cidx], add=True)
        return ()

    lax.fori_loop(0, NBATCH, body, (), unroll=False)
    plsc.subcore_barrier()
    dst = pl.multiple_of(c * ACC_ROWS + s * RPT, 8)
    pltpu.sync_copy(acc.at[rows], bounce)
    pltpu.sync_copy(bounce, out.at[pl.ds(dst, RPT)])


def _sc_deg(col_p, z1):
    mesh = plsc.VectorSubcoreMesh(core_axis_name="c", subcore_axis_name="s")
    return pl.kernel(
        _deg_body,
        out_type=jax.ShapeDtypeStruct((NC * ACC_ROWS,), F32),
        mesh=mesh,
        scratch_types=[
            pltpu.VMEM((EB,), jnp.int32),
            pltpu.VMEM((EB,), F32),
            pltpu.VMEM((RPT,), F32),
            pltpu.VMEM_SHARED((ACC_ROWS,), F32),
            pltpu.SemaphoreType.DMA,
        ],
        name="sc_deg",
    )(col_p, z1)


def _make_sc_pass(widths, name):
    n_ch = len(widths)
    uniq_w = sorted(set(widths), reverse=True)

    def body(*refs):
        tables = refs[:n_ch]
        rowp, colp = refs[n_ch], refs[n_ch + 1]
        zrefs = {w: refs[n_ch + 2 + i] for i, w in enumerate(uniq_w)}
        outs = refs[n_ch + 2 + len(uniq_w):n_ch + 2 + len(uniq_w) + n_ch]
        sc = n_ch + 2 + len(uniq_w) + n_ch
        ridx, cidx = refs[sc], refs[sc + 1]
        rbufs = {w: refs[sc + 2 + i] for i, w in enumerate(uniq_w)}
        bbufs = {w: refs[sc + 2 + len(uniq_w) + i] for i, w in enumerate(uniq_w)}
        accs = {w: refs[sc + 2 + 2 * len(uniq_w) + i]
                for i, w in enumerate(uniq_w)}
        sem = refs[sc + 2 + 3 * len(uniq_w)]

        c = lax.axis_index("c")
        s = lax.axis_index("s")
        base = (c * NS + s) * EPW
        for i, w in enumerate(widths):
            acc, bbuf = accs[w], bbufs[w]
            pltpu.sync_copy(zrefs[w].at[pl.ds(0, RPSUB)], bbuf)
            for j in range(RPT // RPSUB):
                pltpu.sync_copy(bbuf, acc.at[pl.ds(s * RPT + j * RPSUB, RPSUB)])
            plsc.subcore_barrier()
            _edge_loop(rowp, colp, tables[i], acc, ridx, cidx, rbufs[w], sem, base)
            plsc.subcore_barrier()
            for j in range(RPT // RPSUB):
                rj = pl.ds(s * RPT + j * RPSUB, RPSUB)
                pltpu.sync_copy(acc.at[rj], bbuf)
                pltpu.sync_copy(bbuf, outs[i].at[c, rj])
            plsc.subcore_barrier()

    mesh = plsc.VectorSubcoreMesh(core_axis_name="c", subcore_axis_name="s")
    scratch = [pltpu.VMEM((EB,), jnp.int32), pltpu.VMEM((EB,), jnp.int32)]
    scratch += [pltpu.VMEM((EB, w), F32) for w in uniq_w]
    scratch += [pltpu.VMEM((RPSUB, w), F32) for w in uniq_w]
    scratch += [pltpu.VMEM_SHARED((ACC_ROWS, w), F32) for w in uniq_w]
    scratch += [pltpu.SemaphoreType.DMA]
    out_type = tuple(jax.ShapeDtypeStruct((NC, ACC_ROWS, w), F32) for w in widths)

    return pl.kernel(body, out_type=out_type, mesh=mesh,
                     scratch_types=scratch, name=name)


def _tc_a_body(x_ref, degp_ref, GW1_ref, EW1_ref, Eb1_ref,
               t0_ref, t1_ref, t2_ref, t3_ref, dis_ref):
    x = x_ref[...]
    deg = degp_ref[:, 0] + degp_ref[:, 1] + 1.0
    disc = lax.rsqrt(deg)[:, None]
    dis_ref[...] = disc
    g1 = disc * _matmul_t(x, GW1_ref[...])
    t3_ref[...] = jnp.concatenate(
        [g1, jnp.zeros((g1.shape[0], HID - GATE_HID), F32)], axis=1)
    for i, k in enumerate(CURVS):
        xm = _expmap0(x, k)
        xt = _kappa_dense(xm, EW1_ref[i], Eb1_ref[i], k)
        [t0_ref, t1_ref, t2_ref][i][...] = disc * xt


def _tc_a(x, degp, GW1, EW1, Eb1):
    full = lambda a: pl.BlockSpec(a.shape, lambda i: (0,) * a.ndim)
    return pl.pallas_call(
        _tc_a_body,
        grid=(NGRID,),
        in_specs=[
            pl.BlockSpec((RB, IN_DIM), lambda i: (i, 0)),
            pl.BlockSpec((RB, NC), lambda i: (i, 0)),
            full(GW1), full(EW1), full(Eb1),
        ],
        out_specs=[
            pl.BlockSpec((RB, HID), lambda i: (i, 0)),
            pl.BlockSpec((RB, HID), lambda i: (i, 0)),
            pl.BlockSpec((RB, HID), lambda i: (i, 0)),
            pl.BlockSpec((RB, HID), lambda i: (i, 0)),
            pl.BlockSpec((RB, 1), lambda i: (i, 0)),
        ],
        out_shape=[
            jax.ShapeDtypeStruct((N, HID), F32),
            jax.ShapeDtypeStruct((N, HID), F32),
            jax.ShapeDtypeStruct((N, HID), F32),
            jax.ShapeDtypeStruct((N, HID), F32),
            jax.ShapeDtypeStruct((N, 1), F32),
        ],
        name="tc_a",
    )(x, degp, GW1, EW1, Eb1)


def _tc_b_body(a0_ref, a1_ref, a2_ref, a3_ref, t0_ref, t1_ref, t2_ref, t3_ref,
               dis_ref, Gb1_ref, GW2_ref, EW2_ref, Eb2_ref, u0_ref, u1_ref):
    disc = dis_ref[...]
    og = disc * (a3_ref[0, :, :GATE_HID] + a3_ref[1, :, :GATE_HID] +
                 t3_ref[:, :GATE_HID])
    h1 = jax.nn.relu(og + Gb1_ref[...][None, :])
    fg = disc * _matmul_t(h1, GW2_ref[...])
    es = []
    for i, k in enumerate(CURVS):
        a = [a0_ref, a1_ref, a2_ref][i]
        t = [t0_ref, t1_ref, t2_ref][i]
        o = disc * (a[0] + a[1] + t[...])
        xm1 = _expmap0(o, k)
        es.append(disc * _kappa_dense(xm1, EW2_ref[i], Eb2_ref[i], k))
    u0_ref[...] = jnp.concatenate([fg, es[0]], axis=1)
    u1_ref[...] = jnp.concatenate([es[1], es[2]], axis=1)


def _tc_b(a1s, ts, dis, Gb1, GW2, EW2, Eb2):
    full = lambda a: pl.BlockSpec(a.shape, lambda i: (0,) * a.ndim)
    a1s = [a[:, :N, :] for a in a1s]
    specs = [pl.BlockSpec((NC, RB, HID), lambda i: (0, i, 0))] * 4
    specs += [pl.BlockSpec((RB, HID), lambda i: (i, 0))] * 4
    specs += [pl.BlockSpec((RB, 1), lambda i: (i, 0))]
    specs += [full(Gb1), full(GW2), full(EW2), full(Eb2)]
    return pl.pallas_call(
        _tc_b_body,
        grid=(NGRID,),
        in_specs=specs,
        out_specs=[pl.BlockSpec((RB, 2 * OUT), lambda i: (i, 0))] * 2,
        out_shape=[jax.ShapeDtypeStruct((N, 2 * OUT), F32)] * 2,
        name="tc_b",
    )(*a1s, *ts, dis, Gb1, GW2, EW2, Eb2)


def _tc_c_body(a0_ref, a1_ref, u0_ref, u1_ref, dis_ref, Gb2_ref, batch_ref,
               u_ref, tau_ref, out_ref, zacc, cacc):
    i = pl.program_id(0)
    disc = dis_ref[...]
    o0 = disc * (a0_ref[0] + a0_ref[1] + u0_ref[...])
    o1 = disc * (a1_ref[0] + a1_ref[1] + u1_ref[...])
    h2 = jax.nn.relu(o0[:, :OUT] + Gb2_ref[...][None, :])
    zs = [h2]
    for idx, k in enumerate(CURVS):
        o = [o0[:, OUT:], o1[:, :OUT], o1[:, OUT:]][idx]
        zs.append(_logmap0(_expmap0(o, k), k))
    cat = jnp.concatenate(zs, axis=1)
    b = batch_ref[...][:, 0]
    gid = lax.broadcasted_iota(jnp.int32, (NG, RB), 0)
    P = (b[None, :] == gid).astype(F32)
    zpart = lax.dot_general(P, cat, (((1,), (0,)), ((), ())),
                            preferred_element_type=F32)
    cpart = jnp.sum(P, axis=1)

    @pl.when(i == 0)
    def _():
        zacc[...] = jnp.zeros_like(zacc)
        cacc[...] = jnp.zeros_like(cacc)
        out_ref[...] = jnp.zeros_like(out_ref)

    zacc[...] += zpart
    cacc[...] += cpart

    @pl.when(i == NGRID - 1)
    def _():
        cnt = jnp.clip(cacc[...], 1.0, None)[:, None]
        Z = zacc[...] / cnt
        hg = Z[:, :OUT]
        tau_raw = tau_ref[...]
        tau = jnp.clip(jnp.maximum(tau_raw, 0.0) +
                       jnp.log1p(jnp.exp(-jnp.abs(tau_raw))) + TAU_MIN,
                       TAU_MIN, TAU_MAX)
        ds = []
        for idx, k in enumerate(CURVS):
            zz = _expmap0(hg, k)
            yy = _expmap0(jnp.broadcast_to(u_ref[idx], hg.shape), k)
            ds.append(_mdist(zz, yy, k))
        d = jnp.stack(ds, axis=-1)
        lg = -d / tau[None, :]
        m = jnp.max(lg, axis=1, keepdims=True)
        e = jnp.exp(lg - m)
        w = e / jnp.sum(e, axis=1, keepdims=True)
        res = jnp.zeros((NG, OUT), F32)
        for idx in range(3):
            res = res + w[:, idx:idx + 1] * Z[:, OUT * (idx + 1):OUT * (idx + 2)]
        out_ref[...] = res


def _tc_c(a2s, us, dis, Gb2, batch, u, tau_raw):
    full = lambda a: pl.BlockSpec(a.shape, lambda i: (0,) * a.ndim)
    a2s = [a[:, :N, :] for a in a2s]
    specs = [pl.BlockSpec((NC, RB, 2 * OUT), lambda i: (0, i, 0))] * 2
    specs += [pl.BlockSpec((RB, 2 * OUT), lambda i: (i, 0))] * 2
    specs += [pl.BlockSpec((RB, 1), lambda i: (i, 0))]
    specs += [full(Gb2)]
    specs += [pl.BlockSpec((RB, 1), lambda i: (i, 0))]
    specs += [full(u), full(tau_raw)]
    return pl.pallas_call(
        _tc_c_body,
        grid=(NGRID,),
        in_specs=specs,
        out_specs=pl.BlockSpec((NG, OUT), lambda i: (0, 0)),
        out_shape=jax.ShapeDtypeStruct((NG, OUT), F32),
        scratch_shapes=[pltpu.VMEM((NG, 4 * OUT), F32), pltpu.VMEM((NG,), F32)],
        name="tc_c",
    )(*a2s, *us, dis, Gb2, batch, u, tau_raw)


def kernel(x, edge_index, batch, EW1, Eb1, EW2, Eb2, GW1, Gb1, GW2, Gb2, u,
           tau_raw):
    x = x.astype(F32)
    pad = EP - E
    row_p = jnp.concatenate([edge_index[0], jnp.zeros((pad,), jnp.int32)])
    col_p = jnp.concatenate([edge_index[1], jnp.full((pad,), DUMMY, jnp.int32)])
    z1 = jnp.zeros((ACC_ROWS,), F32)
    z128 = jnp.zeros((ACC_ROWS, HID), F32)

    degp = _sc_deg(col_p, z1).reshape(NC, ACC_ROWS)
    t0, t1, t2, t3, dis = _tc_a(x, degp[:, :N].T, GW1, EW1, Eb1)
    pass1 = _make_sc_pass((HID, HID, HID, HID), "sc_pass1")
    a10, a11, a12, a13 = pass1(t0, t1, t2, t3, row_p, col_p, z128)
    u0, u1 = _tc_b([a10, a11, a12, a13], [t0, t1, t2, t3], dis, Gb1, GW2, EW2,
                   Eb2)
    pass2 = _make_sc_pass((2 * OUT, 2 * OUT), "sc_pass2")
    a20, a21 = pass2(u0, u1, row_p, col_p, z128)
    return _tc_c([a20, a21], [u0, u1], dis, Gb2, batch[:, None], u, tau_raw)


# speedup vs baseline: 9.3335x; 1.3587x over previous
import functools
import numpy as np
import jax
import jax.numpy as jnp
from jax import lax
from jax.experimental import pallas as pl
from jax.experimental.pallas import tpu as pltpu
from jax.experimental.pallas import tpu_sc as plsc

N = 10000
E = 320000
IN_DIM = 128
HID = 128
OUT = 64
NG = 64
GATE_HID = 32
CURVS = (-1.0, 0.0, 1.0)
TAU_MIN = 0.05
TAU_MAX = 10.0
EPS = 1e-15

F32 = jnp.float32

NC = 2
NS = 16
NW = NC * NS
EB = 128
NBATCH = -(-E // (EB * NW))
EPW = NBATCH * EB
EP = EPW * NW
ACC_ROWS = 10240
RPT = ACC_ROWS // NS
RPSUB = 80
DUMMY = N

RB = 2000
NGRID = N // RB


def _norm(v):
    return jnp.sqrt(jnp.clip(jnp.sum(v * v, axis=-1, keepdims=True), EPS, None))


def _atan_pos(z):
    t = z
    for _ in range(3):
        t = t / (1.0 + jnp.sqrt(1.0 + t * t))
    t2 = t * t
    p = t * (1.0 + t2 * (-1.0 / 3.0 + t2 * (0.2 + t2 * (-1.0 / 7.0 + t2 / 9.0))))
    return 8.0 * p


def _atanh(z):
    return 0.5 * (jnp.log1p(z) - jnp.log1p(-z))


def _tan_k(t, k):
    if k > 0:
        sk = np.sqrt(k)
        return jnp.tan(sk * t) / sk
    if k < 0:
        sk = np.sqrt(-k)
        return jnp.tanh(sk * t) / sk
    return t


def _artan_k(t, k):
    if k > 0:
        sk = np.sqrt(k)
        return _atan_pos(sk * t) / sk
    if k < 0:
        sk = np.sqrt(-k)
        return _atanh(jnp.clip(sk * t, -1.0 + 1e-7, 1.0 - 1e-7)) / sk
    return t


def _project(v, k):
    if k < 0:
        maxn = (1.0 - 1e-3) / np.sqrt(-k)
        n = _norm(v)
        return jnp.where(n > maxn, v / n * maxn, v)
    return v


def _expmap0(v, k):
    n = _norm(v)
    return _project(_tan_k(n, k) * v / n, k)


def _logmap0(v, k):
    n = _norm(v)
    return _artan_k(n, k) * v / n


def _mobius_add(x, y, k):
    x2 = jnp.sum(x * x, -1, keepdims=True)
    y2 = jnp.sum(y * y, -1, keepdims=True)
    xy = jnp.sum(x * y, -1, keepdims=True)
    num = (1.0 - 2.0 * k * xy - k * y2) * x + (1.0 + k * x2) * y
    den = 1.0 - 2.0 * k * xy + k * k * x2 * y2
    den = jnp.where(jnp.abs(den) < EPS, EPS, den)
    return num / den


def _mdist(x, y, k):
    return 2.0 * _artan_k(jnp.squeeze(_norm(_mobius_add(-x, y, k)), -1), k)


def _matmul_t(a, w):
    return lax.dot_general(a, w, (((1,), (1,)), ((), ())),
                           preferred_element_type=F32)


def _kappa_dense(o, W, b, k):
    lm = _logmap0(o, k)
    y = _matmul_t(lm, W)
    xl = _expmap0(y, k)
    kb = _expmap0(b[None, :], k)
    xl = _project(_mobius_add(xl, kb, k), k)
    return _logmap0(xl, k)


def _agg_chunk(tbl, acc, rowp, colp, base, ribufs, cibufs, rbufs,
               gsems, ssems, isems):
    def idx_load(b, i):
        off = pl.multiple_of(base + b * EB, EB)
        r = pltpu.async_copy(rowp.at[pl.ds(off, EB)], ribufs[i], isems[i])
        c = pltpu.async_copy(colp.at[pl.ds(off, EB)], cibufs[i], isems[i])
        return (r, c)

    gd = [None, None]
    sd = [None, None]
    isd = [None, None, None]
    isd[0] = idx_load(0, 0)
    if NBATCH > 1:
        isd[1] = idx_load(1, 1)
    for d in isd[0]:
        d.wait()
    gd[0] = pltpu.async_copy(tbl.at[ribufs[0]], rbufs[0], gsems[0])
    for b in range(NBATCH):
        p = b & 1
        q = 1 - p
        i0, i1, i2 = b % 3, (b + 1) % 3, (b + 2) % 3
        if b + 1 < NBATCH:
            if sd[q] is not None:
                sd[q].wait()
            for d in isd[i1]:
                d.wait()
            gd[q] = pltpu.async_copy(tbl.at[ribufs[i1]], rbufs[q], gsems[q])
            if b + 2 < NBATCH:
                isd[i2] = idx_load(b + 2, i2)
        gd[p].wait()
        sd[p] = pltpu.async_copy(rbufs[p], acc.at[cibufs[i0]], ssems[p],
                                 add=True)
    for d in sd:
        if d is not None:
            d.wait()


def _deg_body(colp, z1, out, cidx, ones_v, bounce, acc, sem):
    c = lax.axis_index("c")
    s = lax.axis_index("s")
    base = (c * NS + s) * EPW
    for j in range(EB // 16):
        ones_v[pl.ds(j * 16, 16)] = jnp.ones((16,), F32)
    rows = pl.ds(s * RPT, RPT)
    pltpu.sync_copy(z1.at[rows], bounce)
    pltpu.sync_copy(bounce, acc.at[rows])
    plsc.subcore_barrier()

    def body(b, _):
        off = pl.multiple_of(base + b * EB, EB)
        pltpu.sync_copy(colp.at[pl.ds(off, EB)], cidx)
        pltpu.sync_copy(ones_v, acc.at[cidx], add=True)
        return ()

    lax.fori_loop(0, NBATCH, body, (), unroll=False)
    plsc.subcore_barrier()
    dst = pl.multiple_of(c * ACC_ROWS + s * RPT, 8)
    pltpu.sync_copy(acc.at[rows], bounce)
    pltpu.sync_copy(bounce, out.at[pl.ds(dst, RPT)])


def _sc_deg(col_p, z1):
    mesh = plsc.VectorSubcoreMesh(core_axis_name="c", subcore_axis_name="s")
    return pl.kernel(
        _deg_body,
        out_type=jax.ShapeDtypeStruct((NC * ACC_ROWS,), F32),
        mesh=mesh,
        scratch_types=[
            pltpu.VMEM((EB,), jnp.int32),
            pltpu.VMEM((EB,), F32),
            pltpu.VMEM((RPT,), F32),
            pltpu.VMEM_SHARED((ACC_ROWS,), F32),
            pltpu.SemaphoreType.DMA,
        ],
        name="sc_deg",
    )(col_p, z1)


def _make_sc_pass(widths, name):
    n_ch = len(widths)
    uniq_w = sorted(set(widths), reverse=True)

    def body(*refs):
        tables = refs[:n_ch]
        rowp, colp = refs[n_ch], refs[n_ch + 1]
        zrefs = {w: refs[n_ch + 2 + i] for i, w in enumerate(uniq_w)}
        outs = refs[n_ch + 2 + len(uniq_w):n_ch + 2 + len(uniq_w) + n_ch]
        sc = n_ch + 2 + len(uniq_w) + n_ch
        ribufs = refs[sc:sc + 3]
        cibufs = refs[sc + 3:sc + 6]
        rbufs = {w: (refs[sc + 6 + 2 * i], refs[sc + 6 + 2 * i + 1])
                 for i, w in enumerate(uniq_w)}
        nb = sc + 6 + 2 * len(uniq_w)
        bbufs = {w: refs[nb + i] for i, w in enumerate(uniq_w)}
        accs = {w: refs[nb + len(uniq_w) + i] for i, w in enumerate(uniq_w)}
        nse = nb + 2 * len(uniq_w)
        gsems = refs[nse:nse + 2]
        ssems = refs[nse + 2:nse + 4]
        isems = refs[nse + 4:nse + 7]

        c = lax.axis_index("c")
        s = lax.axis_index("s")
        base = (c * NS + s) * EPW
        for i, w in enumerate(widths):
            acc, bbuf = accs[w], bbufs[w]
            pltpu.sync_copy(zrefs[w].at[pl.ds(0, RPSUB)], bbuf)
            for j in range(RPT // RPSUB):
                pltpu.sync_copy(bbuf, acc.at[pl.ds(s * RPT + j * RPSUB, RPSUB)])
            plsc.subcore_barrier()
            _agg_chunk(tables[i], acc, rowp, colp, base, ribufs, cibufs,
                       rbufs[w], gsems, ssems, isems)
            plsc.subcore_barrier()
            for j in range(RPT // RPSUB):
                rj = pl.ds(s * RPT + j * RPSUB, RPSUB)
                pltpu.sync_copy(acc.at[rj], bbuf)
                pltpu.sync_copy(bbuf, outs[i].at[c, rj])
            plsc.subcore_barrier()

    mesh = plsc.VectorSubcoreMesh(core_axis_name="c", subcore_axis_name="s")
    scratch = [pltpu.VMEM((EB,), jnp.int32) for _ in range(6)]
    for w in uniq_w:
        scratch += [pltpu.VMEM((EB, w), F32), pltpu.VMEM((EB, w), F32)]
    scratch += [pltpu.VMEM((RPSUB, w), F32) for w in uniq_w]
    scratch += [pltpu.VMEM_SHARED((ACC_ROWS, w), F32) for w in uniq_w]
    scratch += [pltpu.SemaphoreType.DMA] * 7
    out_type = tuple(jax.ShapeDtypeStruct((NC, ACC_ROWS, w), F32) for w in widths)

    return pl.kernel(body, out_type=out_type, mesh=mesh,
                     scratch_types=scratch, name=name)


def _tc_a_body(x_ref, degp_ref, GW1_ref, EW1_ref, Eb1_ref,
               t0_ref, t1_ref, t2_ref, t3_ref, dis_ref):
    x = x_ref[...]
    deg = degp_ref[:, 0] + degp_ref[:, 1] + 1.0
    disc = lax.rsqrt(deg)[:, None]
    dis_ref[...] = disc
    g1 = disc * _matmul_t(x, GW1_ref[...])
    t3_ref[...] = jnp.concatenate(
        [g1, jnp.zeros((g1.shape[0], HID - GATE_HID), F32)], axis=1)
    for i, k in enumerate(CURVS):
        xm = _expmap0(x, k)
        xt = _kappa_dense(xm, EW1_ref[i], Eb1_ref[i], k)
        [t0_ref, t1_ref, t2_ref][i][...] = disc * xt


def _tc_a(x, degp, GW1, EW1, Eb1):
    full = lambda a: pl.BlockSpec(a.shape, lambda i: (0,) * a.ndim)
    return pl.pallas_call(
        _tc_a_body,
        grid=(NGRID,),
        in_specs=[
            pl.BlockSpec((RB, IN_DIM), lambda i: (i, 0)),
            pl.BlockSpec((RB, NC), lambda i: (i, 0)),
            full(GW1), full(EW1), full(Eb1),
        ],
        out_specs=[
            pl.BlockSpec((RB, HID), lambda i: (i, 0)),
            pl.BlockSpec((RB, HID), lambda i: (i, 0)),
            pl.BlockSpec((RB, HID), lambda i: (i, 0)),
            pl.BlockSpec((RB, HID), lambda i: (i, 0)),
            pl.BlockSpec((RB, 1), lambda i: (i, 0)),
        ],
        out_shape=[
            jax.ShapeDtypeStruct((N, HID), F32),
            jax.ShapeDtypeStruct((N, HID), F32),
            jax.ShapeDtypeStruct((N, HID), F32),
            jax.ShapeDtypeStruct((N, HID), F32),
            jax.ShapeDtypeStruct((N, 1), F32),
        ],
        name="tc_a",
    )(x, degp, GW1, EW1, Eb1)


def _tc_b_body(a0_ref, a1_ref, a2_ref, a3_ref, t0_ref, t1_ref, t2_ref, t3_ref,
               dis_ref, Gb1_ref, GW2_ref, EW2_ref, Eb2_ref, u0_ref, u1_ref):
    disc = dis_ref[...]
    og = disc * (a3_ref[0, :, :GATE_HID] + a3_ref[1, :, :GATE_HID] +
                 t3_ref[:, :GATE_HID])
    h1 = jax.nn.relu(og + Gb1_ref[...][None, :])
    fg = disc * _matmul_t(h1, GW2_ref[...])
    es = []
    for i, k in enumerate(CURVS):
        a = [a0_ref, a1_ref, a2_ref][i]
        t = [t0_ref, t1_ref, t2_ref][i]
        o = disc * (a[0] + a[1] + t[...])
        xm1 = _expmap0(o, k)
        es.append(disc * _kappa_dense(xm1, EW2_ref[i], Eb2_ref[i], k))
    u0_ref[...] = jnp.concatenate([fg, es[0]], axis=1)
    u1_ref[...] = jnp.concatenate([es[1], es[2]], axis=1)


def _tc_b(a1s, ts, dis, Gb1, GW2, EW2, Eb2):
    full = lambda a: pl.BlockSpec(a.shape, lambda i: (0,) * a.ndim)
    a1s = [a[:, :N, :] for a in a1s]
    specs = [pl.BlockSpec((NC, RB, HID), lambda i: (0, i, 0))] * 4
    specs += [pl.BlockSpec((RB, HID), lambda i: (i, 0))] * 4
    specs += [pl.BlockSpec((RB, 1), lambda i: (i, 0))]
    specs += [full(Gb1), full(GW2), full(EW2), full(Eb2)]
    return pl.pallas_call(
        _tc_b_body,
        grid=(NGRID,),
        in_specs=specs,
        out_specs=[pl.BlockSpec((RB, 2 * OUT), lambda i: (i, 0))] * 2,
        out_shape=[jax.ShapeDtypeStruct((N, 2 * OUT), F32)] * 2,
        name="tc_b",
    )(*a1s, *ts, dis, Gb1, GW2, EW2, Eb2)


def _tc_c_body(a0_ref, a1_ref, u0_ref, u1_ref, dis_ref, Gb2_ref, batch_ref,
               u_ref, tau_ref, out_ref, zacc, cacc):
    i = pl.program_id(0)
    disc = dis_ref[...]
    o0 = disc * (a0_ref[0] + a0_ref[1] + u0_ref[...])
    o1 = disc * (a1_ref[0] + a1_ref[1] + u1_ref[...])
    h2 = jax.nn.relu(o0[:, :OUT] + Gb2_ref[...][None, :])
    zs = [h2]
    for idx, k in enumerate(CURVS):
        o = [o0[:, OUT:], o1[:, :OUT], o1[:, OUT:]][idx]
        zs.append(_logmap0(_expmap0(o, k), k))
    cat = jnp.concatenate(zs, axis=1)
    b = batch_ref[...][:, 0]
    gid = lax.broadcasted_iota(jnp.int32, (NG, RB), 0)
    P = (b[None, :] == gid).astype(F32)
    zpart = lax.dot_general(P, cat, (((1,), (0,)), ((), ())),
                            preferred_element_type=F32)
    cpart = jnp.sum(P, axis=1)

    @pl.when(i == 0)
    def _():
        zacc[...] = jnp.zeros_like(zacc)
        cacc[...] = jnp.zeros_like(cacc)
        out_ref[...] = jnp.zeros_like(out_ref)

    zacc[...] += zpart
    cacc[...] += cpart

    @pl.when(i == NGRID - 1)
    def _():
        cnt = jnp.clip(cacc[...], 1.0, None)[:, None]
        Z = zacc[...] / cnt
        hg = Z[:, :OUT]
        tau_raw = tau_ref[...]
        tau = jnp.clip(jnp.maximum(tau_raw, 0.0) +
                       jnp.log1p(jnp.exp(-jnp.abs(tau_raw))) + TAU_MIN,
                       TAU_MIN, TAU_MAX)
        ds = []
        for idx, k in enumerate(CURVS):
            zz = _expmap0(hg, k)
            yy = _expmap0(jnp.broadcast_to(u_ref[idx], hg.shape), k)
            ds.append(_mdist(zz, yy, k))
        d = jnp.stack(ds, axis=-1)
        lg = -d / tau[None, :]
        m = jnp.max(lg, axis=1, keepdims=True)
        e = jnp.exp(lg - m)
        w = e / jnp.sum(e, axis=1, keepdims=True)
        res = jnp.zeros((NG, OUT), F32)
        for idx in range(3):
            res = res + w[:, idx:idx + 1] * Z[:, OUT * (idx + 1):OUT * (idx + 2)]
        out_ref[...] = res


def _tc_c(a2s, us, dis, Gb2, batch, u, tau_raw):
    full = lambda a: pl.BlockSpec(a.shape, lambda i: (0,) * a.ndim)
    a2s = [a[:, :N, :] for a in a2s]
    specs = [pl.BlockSpec((NC, RB, 2 * OUT), lambda i: (0, i, 0))] * 2
    specs += [pl.BlockSpec((RB, 2 * OUT), lambda i: (i, 0))] * 2
    specs += [pl.BlockSpec((RB, 1), lambda i: (i, 0))]
    specs += [full(Gb2)]
    specs += [pl.BlockSpec((RB, 1), lambda i: (i, 0))]
    specs += [full(u), full(tau_raw)]
    return pl.pallas_call(
        _tc_c_body,
        grid=(NGRID,),
        in_specs=specs,
        out_specs=pl.BlockSpec((NG, OUT), lambda i: (0, 0)),
        out_shape=jax.ShapeDtypeStruct((NG, OUT), F32),
        scratch_shapes=[pltpu.VMEM((NG, 4 * OUT), F32), pltpu.VMEM((NG,), F32)],
        name="tc_c",
    )(*a2s, *us, dis, Gb2, batch, u, tau_raw)


def kernel(x, edge_index, batch, EW1, Eb1, EW2, Eb2, GW1, Gb1, GW2, Gb2, u,
           tau_raw):
    x = x.astype(F32)
    pad = EP - E
    row_p = jnp.concatenate([edge_index[0], jnp.zeros((pad,), jnp.int32)])
    col_p = jnp.concatenate([edge_index[1], jnp.full((pad,), DUMMY, jnp.int32)])
    z1 = jnp.zeros((ACC_ROWS,), F32)
    z128 = jnp.zeros((ACC_ROWS, HID), F32)

    degp = _sc_deg(col_p, z1).reshape(NC, ACC_ROWS)
    t0, t1, t2, t3, dis = _tc_a(x, degp[:, :N].T, GW1, EW1, Eb1)
    pass1 = _make_sc_pass((HID, HID, HID, HID), "sc_pass1")
    a10, a11, a12, a13 = pass1(t0, t1, t2, t3, row_p, col_p, z128)
    u0, u1 = _tc_b([a10, a11, a12, a13], [t0, t1, t2, t3], dis, Gb1, GW2, EW2,
                   Eb2)
    pass2 = _make_sc_pass((2 * OUT, 2 * OUT), "sc_pass2")
    a20, a21 = pass2(u0, u1, row_p, col_p, z128)
    return _tc_c([a20, a21], [u0, u1], dis, Gb2, batch[:, None], u, tau_raw)


# speedup vs baseline: 17.5774x; 1.8833x over previous
import functools
import numpy as np
import jax
import jax.numpy as jnp
from jax import lax
from jax.experimental import pallas as pl
from jax.experimental.pallas import tpu as pltpu
from jax.experimental.pallas import tpu_sc as plsc

N = 10000
E = 320000
IN_DIM = 128
HID = 128
OUT = 64
NG = 64
GATE_HID = 32
CURVS = (-1.0, 0.0, 1.0)
TAU_MIN = 0.05
TAU_MAX = 10.0
EPS = 1e-15

F32 = jnp.float32

NC = 2
NS = 16
NW = NC * NS
EB = 128
NBATCH = -(-E // (EB * NW))
EPW = NBATCH * EB
EP = EPW * NW
ACC_ROWS = 10240
RPT = ACC_ROWS // NS
RPSUB = 80
DUMMY = N

RB = 2000
NGRID = N // RB


def _norm(v):
    return jnp.sqrt(jnp.clip(jnp.sum(v * v, axis=-1, keepdims=True), EPS, None))


def _atan_pos(z):
    t = z
    for _ in range(3):
        t = t / (1.0 + jnp.sqrt(1.0 + t * t))
    t2 = t * t
    p = t * (1.0 + t2 * (-1.0 / 3.0 + t2 * (0.2 + t2 * (-1.0 / 7.0 + t2 / 9.0))))
    return 8.0 * p


def _atanh(z):
    return 0.5 * (jnp.log1p(z) - jnp.log1p(-z))


def _tan_k(t, k):
    if k > 0:
        sk = np.sqrt(k)
        return jnp.tan(sk * t) / sk
    if k < 0:
        sk = np.sqrt(-k)
        return jnp.tanh(sk * t) / sk
    return t


def _artan_k(t, k):
    if k > 0:
        sk = np.sqrt(k)
        return _atan_pos(sk * t) / sk
    if k < 0:
        sk = np.sqrt(-k)
        return _atanh(jnp.clip(sk * t, -1.0 + 1e-7, 1.0 - 1e-7)) / sk
    return t


def _project(v, k):
    if k < 0:
        maxn = (1.0 - 1e-3) / np.sqrt(-k)
        n = _norm(v)
        return jnp.where(n > maxn, v / n * maxn, v)
    return v


def _expmap0(v, k):
    n = _norm(v)
    return _project(_tan_k(n, k) * v / n, k)


def _logmap0(v, k):
    n = _norm(v)
    return _artan_k(n, k) * v / n


def _mobius_add(x, y, k):
    x2 = jnp.sum(x * x, -1, keepdims=True)
    y2 = jnp.sum(y * y, -1, keepdims=True)
    xy = jnp.sum(x * y, -1, keepdims=True)
    num = (1.0 - 2.0 * k * xy - k * y2) * x + (1.0 + k * x2) * y
    den = 1.0 - 2.0 * k * xy + k * k * x2 * y2
    den = jnp.where(jnp.abs(den) < EPS, EPS, den)
    return num / den


def _mdist(x, y, k):
    return 2.0 * _artan_k(jnp.squeeze(_norm(_mobius_add(-x, y, k)), -1), k)


def _matmul_t(a, w):
    return lax.dot_general(a, w, (((1,), (1,)), ((), ())),
                           preferred_element_type=F32)


def _kappa_dense(o, W, b, k):
    lm = _logmap0(o, k)
    y = _matmul_t(lm, W)
    xl = _expmap0(y, k)
    kb = _expmap0(b[None, :], k)
    xl = _project(_mobius_add(xl, kb, k), k)
    return _logmap0(xl, k)


def _agg_chunk(tbl, acc, rowp, colp, base, ribufs, cibufs, rbufs,
               gsems, ssems, isems):
    def idx_load(b, i):
        off = pl.multiple_of(base + b * EB, EB)
        r = pltpu.async_copy(rowp.at[pl.ds(off, EB)], ribufs[i], isems[i])
        c = pltpu.async_copy(colp.at[pl.ds(off, EB)], cibufs[i], isems[i])
        return (r, c)

    gd = [None, None]
    sd = [None, None]
    isd = [None, None, None]
    isd[0] = idx_load(0, 0)
    if NBATCH > 1:
        isd[1] = idx_load(1, 1)
    for d in isd[0]:
        d.wait()
    gd[0] = pltpu.async_copy(tbl.at[ribufs[0]], rbufs[0], gsems[0])
    for b in range(NBATCH):
        p = b & 1
        q = 1 - p
        i0, i1, i2 = b % 3, (b + 1) % 3, (b + 2) % 3
        if b + 1 < NBATCH:
            if sd[q] is not None:
                sd[q].wait()
            for d in isd[i1]:
                d.wait()
            gd[q] = pltpu.async_copy(tbl.at[ribufs[i1]], rbufs[q], gsems[q])
            if b + 2 < NBATCH:
                isd[i2] = idx_load(b + 2, i2)
        gd[p].wait()
        sd[p] = pltpu.async_copy(rbufs[p], acc.at[cibufs[i0]], ssems[p],
                                 add=True)
    for d in sd:
        if d is not None:
            d.wait()


def _deg_body(colp, z1, out, cidx, ones_v, bounce, acc, sem):
    c = lax.axis_index("c")
    s = lax.axis_index("s")
    base = (c * NS + s) * EPW
    for j in range(EB // 16):
        ones_v[pl.ds(j * 16, 16)] = jnp.ones((16,), F32)
    rows = pl.ds(s * RPT, RPT)
    pltpu.sync_copy(z1.at[rows], bounce)
    pltpu.sync_copy(bounce, acc.at[rows])
    plsc.subcore_barrier()

    def body(b, _):
        off = pl.multiple_of(base + b * EB, EB)
        pltpu.sync_copy(colp.at[pl.ds(off, EB)], cidx)
        pltpu.sync_copy(ones_v, acc.at[cidx], add=True)
        return ()

    lax.fori_loop(0, NBATCH, body, (), unroll=False)
    plsc.subcore_barrier()
    dst = pl.multiple_of(c * ACC_ROWS + s * RPT, 8)
    pltpu.sync_copy(acc.at[rows], bounce)
    pltpu.sync_copy(bounce, out.at[pl.ds(dst, RPT)])


def _sc_deg(col_p, z1):
    mesh = plsc.VectorSubcoreMesh(core_axis_name="c", subcore_axis_name="s")
    return pl.kernel(
        _deg_body,
        out_type=jax.ShapeDtypeStruct((NC * ACC_ROWS,), F32),
        mesh=mesh,
        scratch_types=[
            pltpu.VMEM((EB,), jnp.int32),
            pltpu.VMEM((EB,), F32),
            pltpu.VMEM((RPT,), F32),
            pltpu.VMEM_SHARED((ACC_ROWS,), F32),
            pltpu.SemaphoreType.DMA,
        ],
        name="sc_deg",
    )(col_p, z1)


def _make_sc_pass(widths, name):
    n_ch = len(widths)
    uniq_w = sorted(set(widths), reverse=True)

    def body(*refs):
        tables = refs[:n_ch]
        rowp, colp = refs[n_ch], refs[n_ch + 1]
        zrefs = {w: refs[n_ch + 2 + i] for i, w in enumerate(uniq_w)}
        outs = refs[n_ch + 2 + len(uniq_w):n_ch + 2 + len(uniq_w) + n_ch]
        sc = n_ch + 2 + len(uniq_w) + n_ch
        ribufs = refs[sc:sc + 3]
        cibufs = refs[sc + 3:sc + 6]
        rbufs = {w: (refs[sc + 6 + 2 * i], refs[sc + 6 + 2 * i + 1])
                 for i, w in enumerate(uniq_w)}
        nb = sc + 6 + 2 * len(uniq_w)
        bbufs = {w: refs[nb + i] for i, w in enumerate(uniq_w)}
        accs = {w: refs[nb + len(uniq_w) + i] for i, w in enumerate(uniq_w)}
        nse = nb + 2 * len(uniq_w)
        gsems = refs[nse:nse + 2]
        ssems = refs[nse + 2:nse + 4]
        isems = refs[nse + 4:nse + 7]

        c = lax.axis_index("c")
        s = lax.axis_index("s")
        base = (c * NS + s) * EPW
        for i, w in enumerate(widths):
            acc, bbuf = accs[w], bbufs[w]
            pltpu.sync_copy(zrefs[w].at[pl.ds(0, RPSUB)], bbuf)
            for j in range(RPT // RPSUB):
                pltpu.sync_copy(bbuf, acc.at[pl.ds(s * RPT + j * RPSUB, RPSUB)])
            plsc.subcore_barrier()
            _agg_chunk(tables[i], acc, rowp, colp, base, ribufs, cibufs,
                       rbufs[w], gsems, ssems, isems)
            plsc.subcore_barrier()
            for j in range(RPT // RPSUB):
                rj = pl.ds(s * RPT + j * RPSUB, RPSUB)
                pltpu.sync_copy(acc.at[rj], bbuf)
                pltpu.sync_copy(bbuf, outs[i].at[c, rj])
            plsc.subcore_barrier()

    mesh = plsc.VectorSubcoreMesh(core_axis_name="c", subcore_axis_name="s")
    scratch = [pltpu.VMEM((EB,), jnp.int32) for _ in range(6)]
    for w in uniq_w:
        scratch += [pltpu.VMEM((EB, w), F32), pltpu.VMEM((EB, w), F32)]
    scratch += [pltpu.VMEM((RPSUB, w), F32) for w in uniq_w]
    scratch += [pltpu.VMEM_SHARED((ACC_ROWS, w), F32) for w in uniq_w]
    scratch += [pltpu.SemaphoreType.DMA] * 7
    out_type = tuple(jax.ShapeDtypeStruct((NC, ACC_ROWS, w), F32) for w in widths)

    return pl.kernel(body, out_type=out_type, mesh=mesh,
                     scratch_types=scratch, name=name)


def _tc_a_body(x_ref, degp_ref, GW1_ref, EW1_ref, Eb1_ref,
               t0_ref, t1_ref, t2_ref, t3_ref, dis_ref):
    x = x_ref[...]
    deg = degp_ref[:, 0] + degp_ref[:, 1] + 1.0
    disc = lax.rsqrt(deg)[:, None]
    dis_ref[...] = disc
    g1 = disc * _matmul_t(x, GW1_ref[...])
    t3_ref[...] = jnp.concatenate(
        [g1, jnp.zeros((g1.shape[0], HID - GATE_HID), F32)], axis=1)
    for i, k in enumerate(CURVS):
        xm = _expmap0(x, k)
        xt = _kappa_dense(xm, EW1_ref[i], Eb1_ref[i], k)
        [t0_ref, t1_ref, t2_ref][i][...] = disc * xt


def _tc_a(x, degp, GW1, EW1, Eb1):
    full = lambda a: pl.BlockSpec(a.shape, lambda i: (0,) * a.ndim)
    return pl.pallas_call(
        _tc_a_body,
        grid=(NGRID,),
        in_specs=[
            pl.BlockSpec((RB, IN_DIM), lambda i: (i, 0)),
            pl.BlockSpec((RB, NC), lambda i: (i, 0)),
            full(GW1), full(EW1), full(Eb1),
        ],
        out_specs=[
            pl.BlockSpec((RB, HID), lambda i: (i, 0)),
            pl.BlockSpec((RB, HID), lambda i: (i, 0)),
            pl.BlockSpec((RB, HID), lambda i: (i, 0)),
            pl.BlockSpec((RB, HID), lambda i: (i, 0)),
            pl.BlockSpec((RB, 1), lambda i: (i, 0)),
        ],
        out_shape=[
            jax.ShapeDtypeStruct((N, HID), F32),
            jax.ShapeDtypeStruct((N, HID), F32),
            jax.ShapeDtypeStruct((N, HID), F32),
            jax.ShapeDtypeStruct((N, HID), F32),
            jax.ShapeDtypeStruct((N, 1), F32),
        ],
        name="tc_a",
    )(x, degp, GW1, EW1, Eb1)


def _tc_b_body(a0_ref, a1_ref, a2_ref, a3_ref, t0_ref, t1_ref, t2_ref, t3_ref,
               dis_ref, Gb1_ref, GW2_ref, EW2_ref, Eb2_ref, u0_ref, u1_ref):
    disc = dis_ref[...]
    og = disc * (a3_ref[0, :, :GATE_HID] + a3_ref[1, :, :GATE_HID] +
                 t3_ref[:, :GATE_HID])
    h1 = jax.nn.relu(og + Gb1_ref[...][None, :])
    fg = disc * _matmul_t(h1, GW2_ref[...])
    es = []
    for i, k in enumerate(CURVS):
        a = [a0_ref, a1_ref, a2_ref][i]
        t = [t0_ref, t1_ref, t2_ref][i]
        o = disc * (a[0] + a[1] + t[...])
        xm1 = _expmap0(o, k)
        es.append(disc * _kappa_dense(xm1, EW2_ref[i], Eb2_ref[i], k))
    u0_ref[...] = jnp.concatenate([fg, es[0]], axis=1)
    u1_ref[...] = jnp.concatenate([es[1], es[2]], axis=1)


def _tc_b(a1s, ts, dis, Gb1, GW2, EW2, Eb2):
    full = lambda a: pl.BlockSpec(a.shape, lambda i: (0,) * a.ndim)
    a1s = [a[:, :N, :] for a in a1s]
    specs = [pl.BlockSpec((NC, RB, HID), lambda i: (0, i, 0))] * 4
    specs += [pl.BlockSpec((RB, HID), lambda i: (i, 0))] * 4
    specs += [pl.BlockSpec((RB, 1), lambda i: (i, 0))]
    specs += [full(Gb1), full(GW2), full(EW2), full(Eb2)]
    return pl.pallas_call(
        _tc_b_body,
        grid=(NGRID,),
        in_specs=specs,
        out_specs=[pl.BlockSpec((RB, 2 * OUT), lambda i: (i, 0))] * 2,
        out_shape=[jax.ShapeDtypeStruct((N, 2 * OUT), F32)] * 2,
        name="tc_b",
    )(*a1s, *ts, dis, Gb1, GW2, EW2, Eb2)


def _tc_c_body(a0_ref, a1_ref, u0_ref, u1_ref, dis_ref, Gb2_ref, batch_ref,
               u_ref, tau_ref, out_ref, zacc, cacc):
    i = pl.program_id(0)
    disc = dis_ref[...]
    o0 = disc * (a0_ref[0] + a0_ref[1] + u0_ref[...])
    o1 = disc * (a1_ref[0] + a1_ref[1] + u1_ref[...])
    h2 = jax.nn.relu(o0[:, :OUT] + Gb2_ref[...][None, :])
    zs = [h2]
    for idx, k in enumerate(CURVS):
        o = [o0[:, OUT:], o1[:, :OUT], o1[:, OUT:]][idx]
        zs.append(_logmap0(_expmap0(o, k), k))
    cat = jnp.concatenate(zs, axis=1)
    b = batch_ref[...][:, 0]
    gid = lax.broadcasted_iota(jnp.int32, (NG, RB), 0)
    P = (b[None, :] == gid).astype(F32)
    zpart = lax.dot_general(P, cat, (((1,), (0,)), ((), ())),
                            preferred_element_type=F32)
    cpart = jnp.sum(P, axis=1)

    @pl.when(i == 0)
    def _():
        zacc[...] = jnp.zeros_like(zacc)
        cacc[...] = jnp.zeros_like(cacc)
        out_ref[...] = jnp.zeros_like(out_ref)

    zacc[...] += zpart
    cacc[...] += cpart

    @pl.when(i == NGRID - 1)
    def _():
        cnt = jnp.clip(cacc[...], 1.0, None)[:, None]
        Z = zacc[...] / cnt
        hg = Z[:, :OUT]
        tau_raw = tau_ref[...]
        tau = jnp.clip(jnp.maximum(tau_raw, 0.0) +
                       jnp.log1p(jnp.exp(-jnp.abs(tau_raw))) + TAU_MIN,
                       TAU_MIN, TAU_MAX)
        ds = []
        for idx, k in enumerate(CURVS):
            zz = _expmap0(hg, k)
            yy = _expmap0(jnp.broadcast_to(u_ref[idx], hg.shape), k)
            ds.append(_mdist(zz, yy, k))
        d = jnp.stack(ds, axis=-1)
        lg = -d / tau[None, :]
        m = jnp.max(lg, axis=1, keepdims=True)
        e = jnp.exp(lg - m)
        w = e / jnp.sum(e, axis=1, keepdims=True)
        res = jnp.zeros((NG, OUT), F32)
        for idx in range(3):
            res = res + w[:, idx:idx + 1] * Z[:, OUT * (idx + 1):OUT * (idx + 2)]
        out_ref[...] = res


def _tc_c(a2s, us, dis, Gb2, batch, u, tau_raw):
    full = lambda a: pl.BlockSpec(a.shape, lambda i: (0,) * a.ndim)
    a2s = [a[:, :N, :] for a in a2s]
    specs = [pl.BlockSpec((NC, RB, 2 * OUT), lambda i: (0, i, 0))] * 2
    specs += [pl.BlockSpec((RB, 2 * OUT), lambda i: (i, 0))] * 2
    specs += [pl.BlockSpec((RB, 1), lambda i: (i, 0))]
    specs += [full(Gb2)]
    specs += [pl.BlockSpec((RB, 1), lambda i: (i, 0))]
    specs += [full(u), full(tau_raw)]
    return pl.pallas_call(
        _tc_c_body,
        grid=(NGRID,),
        in_specs=specs,
        out_specs=pl.BlockSpec((NG, OUT), lambda i: (0, 0)),
        out_shape=jax.ShapeDtypeStruct((NG, OUT), F32),
        scratch_shapes=[pltpu.VMEM((NG, 4 * OUT), F32), pltpu.VMEM((NG,), F32)],
        name="tc_c",
    )(*a2s, *us, dis, Gb2, batch, u, tau_raw)


def kernel(x, edge_index, batch, EW1, Eb1, EW2, Eb2, GW1, Gb1, GW2, Gb2, u,
           tau_raw):
    x = x.astype(F32)
    pad = EP - E
    prow = jnp.arange(pad, dtype=jnp.int32) % N
    pcol = DUMMY + jnp.arange(pad, dtype=jnp.int32) % (ACC_ROWS - N)
    row_p = jnp.concatenate([edge_index[0], prow])
    col_p = jnp.concatenate([edge_index[1], pcol])
    z1 = jnp.zeros((ACC_ROWS,), F32)
    z128 = jnp.zeros((ACC_ROWS, HID), F32)

    degp = _sc_deg(col_p, z1).reshape(NC, ACC_ROWS)
    t0, t1, t2, t3, dis = _tc_a(x, degp[:, :N].T, GW1, EW1, Eb1)
    pass1 = _make_sc_pass((HID, HID, HID, HID), "sc_pass1")
    a10, a11, a12, a13 = pass1(t0, t1, t2, t3, row_p, col_p, z128)
    u0, u1 = _tc_b([a10, a11, a12, a13], [t0, t1, t2, t3], dis, Gb1, GW2, EW2,
                   Eb2)
    pass2 = _make_sc_pass((2 * OUT, 2 * OUT), "sc_pass2")
    a20, a21 = pass2(u0, u1, row_p, col_p, z128)
    return _tc_c([a20, a21], [u0, u1], dis, Gb2, batch[:, None], u, tau_raw)


# speedup vs baseline: 18.6594x; 1.0616x over previous
import functools
import numpy as np
import jax
import jax.numpy as jnp
from jax import lax
from jax.experimental import pallas as pl
from jax.experimental.pallas import tpu as pltpu
from jax.experimental.pallas import tpu_sc as plsc

N = 10000
E = 320000
IN_DIM = 128
HID = 128
OUT = 64
NG = 64
GATE_HID = 32
CURVS = (-1.0, 0.0, 1.0)
TAU_MIN = 0.05
TAU_MAX = 10.0
EPS = 1e-15

F32 = jnp.float32

NC = 2
NS = 16
NW = NC * NS
EB = 128
NBATCH = -(-E // (EB * NW))
EPW = NBATCH * EB
EP = EPW * NW
ACC_ROWS = 10240
RPT = ACC_ROWS // NS
RPSUB = 80
DUMMY = N

RB = 2000
NGRID = N // RB


def _norm(v):
    return jnp.sqrt(jnp.clip(jnp.sum(v * v, axis=-1, keepdims=True), EPS, None))


def _atan_pos(z):
    t = z
    for _ in range(3):
        t = t / (1.0 + jnp.sqrt(1.0 + t * t))
    t2 = t * t
    p = t * (1.0 + t2 * (-1.0 / 3.0 + t2 * (0.2 + t2 * (-1.0 / 7.0 + t2 / 9.0))))
    return 8.0 * p


def _atanh(z):
    return 0.5 * (jnp.log1p(z) - jnp.log1p(-z))


def _tan_k(t, k):
    if k > 0:
        sk = np.sqrt(k)
        return jnp.tan(sk * t) / sk
    if k < 0:
        sk = np.sqrt(-k)
        return jnp.tanh(sk * t) / sk
    return t


def _artan_k(t, k):
    if k > 0:
        sk = np.sqrt(k)
        return _atan_pos(sk * t) / sk
    if k < 0:
        sk = np.sqrt(-k)
        return _atanh(jnp.clip(sk * t, -1.0 + 1e-7, 1.0 - 1e-7)) / sk
    return t


def _project(v, k):
    if k < 0:
        maxn = (1.0 - 1e-3) / np.sqrt(-k)
        n = _norm(v)
        return jnp.where(n > maxn, v / n * maxn, v)
    return v


def _expmap0(v, k):
    n = _norm(v)
    return _project(_tan_k(n, k) * v / n, k)


def _logmap0(v, k):
    n = _norm(v)
    return _artan_k(n, k) * v / n


def _mobius_add(x, y, k):
    x2 = jnp.sum(x * x, -1, keepdims=True)
    y2 = jnp.sum(y * y, -1, keepdims=True)
    xy = jnp.sum(x * y, -1, keepdims=True)
    num = (1.0 - 2.0 * k * xy - k * y2) * x + (1.0 + k * x2) * y
    den = 1.0 - 2.0 * k * xy + k * k * x2 * y2
    den = jnp.where(jnp.abs(den) < EPS, EPS, den)
    return num / den


def _mdist(x, y, k):
    return 2.0 * _artan_k(jnp.squeeze(_norm(_mobius_add(-x, y, k)), -1), k)


def _matmul_t(a, w):
    return lax.dot_general(a, w, (((1,), (1,)), ((), ())),
                           preferred_element_type=F32)


def _kappa_dense(o, W, b, k):
    lm = _logmap0(o, k)
    y = _matmul_t(lm, W)
    xl = _expmap0(y, k)
    kb = _expmap0(b[None, :], k)
    xl = _project(_mobius_add(xl, kb, k), k)
    return _logmap0(xl, k)


def _agg_chunk(tbl, acc, rowp, colp, base, ribufs, cibufs, rbufs,
               gsems, ssems, isems):
    def idx_load(b, i):
        off = pl.multiple_of(base + b * EB, EB)
        r = pltpu.async_copy(rowp.at[pl.ds(off, EB)], ribufs[i], isems[i])
        c = pltpu.async_copy(colp.at[pl.ds(off, EB)], cibufs[i], isems[i])
        return (r, c)

    gd = [None, None]
    sd = [None, None]
    isd = [None, None, None]
    isd[0] = idx_load(0, 0)
    if NBATCH > 1:
        isd[1] = idx_load(1, 1)
    for d in isd[0]:
        d.wait()
    gd[0] = pltpu.async_copy(tbl.at[ribufs[0]], rbufs[0], gsems[0])
    for b in range(NBATCH):
        p = b & 1
        q = 1 - p
        i0, i1, i2 = b % 3, (b + 1) % 3, (b + 2) % 3
        if b + 1 < NBATCH:
            if sd[q] is not None:
                sd[q].wait()
            for d in isd[i1]:
                d.wait()
            gd[q] = pltpu.async_copy(tbl.at[ribufs[i1]], rbufs[q], gsems[q])
            if b + 2 < NBATCH:
                isd[i2] = idx_load(b + 2, i2)
        gd[p].wait()
        sd[p] = pltpu.async_copy(rbufs[p], acc.at[cibufs[i0]], ssems[p],
                                 add=True)
    for d in sd:
        if d is not None:
            d.wait()


def _deg_body(colp, z1, out, c0, c1, c2, c3, ones_v, bounce, acc, s0, s1,
              i0, i1, i2, i3):
    c = lax.axis_index("c")
    s = lax.axis_index("s")
    base = (c * NS + s) * EPW
    cibufs = (c0, c1, c2, c3)
    isems = (i0, i1, i2, i3)
    ssems = (s0, s1)
    for j in range(EB // 16):
        ones_v[pl.ds(j * 16, 16)] = jnp.ones((16,), F32)
    rows = pl.ds(s * RPT, RPT)
    pltpu.sync_copy(z1.at[rows], bounce)
    pltpu.sync_copy(bounce, acc.at[rows])
    plsc.subcore_barrier()

    def idx_load(b, i):
        off = pl.multiple_of(base + b * EB, EB)
        return pltpu.async_copy(colp.at[pl.ds(off, EB)], cibufs[i], isems[i])

    sd = [None, None]
    isd = [None, None, None, None]
    isd[0] = idx_load(0, 0)
    if NBATCH > 1:
        isd[1] = idx_load(1, 1)
    for b in range(NBATCH):
        p = b & 1
        if sd[p] is not None:
            sd[p].wait()
        isd[b % 4].wait()
        sd[p] = pltpu.async_copy(ones_v, acc.at[cibufs[b % 4]], ssems[p],
                                 add=True)
        if b + 2 < NBATCH:
            isd[(b + 2) % 4] = idx_load(b + 2, (b + 2) % 4)
    for d in sd:
        if d is not None:
            d.wait()
    plsc.subcore_barrier()
    dst = pl.multiple_of(c * ACC_ROWS + s * RPT, 8)
    pltpu.sync_copy(acc.at[rows], bounce)
    pltpu.sync_copy(bounce, out.at[pl.ds(dst, RPT)])


def _sc_deg(col_p, z1):
    mesh = plsc.VectorSubcoreMesh(core_axis_name="c", subcore_axis_name="s")
    return pl.kernel(
        _deg_body,
        out_type=jax.ShapeDtypeStruct((NC * ACC_ROWS,), F32),
        mesh=mesh,
        scratch_types=[pltpu.VMEM((EB,), jnp.int32) for _ in range(4)] + [
            pltpu.VMEM((EB,), F32),
            pltpu.VMEM((RPT,), F32),
            pltpu.VMEM_SHARED((ACC_ROWS,), F32),
        ] + [pltpu.SemaphoreType.DMA] * 6,
        name="sc_deg",
    )(col_p, z1)


def _make_sc_pass(widths, name):
    n_ch = len(widths)
    uniq_w = sorted(set(widths), reverse=True)

    def body(*refs):
        tables = refs[:n_ch]
        rowp, colp = refs[n_ch], refs[n_ch + 1]
        zrefs = {w: refs[n_ch + 2 + i] for i, w in enumerate(uniq_w)}
        outs = refs[n_ch + 2 + len(uniq_w):n_ch + 2 + len(uniq_w) + n_ch]
        sc = n_ch + 2 + len(uniq_w) + n_ch
        ribufs = refs[sc:sc + 3]
        cibufs = refs[sc + 3:sc + 6]
        rbufs = {w: (refs[sc + 6 + 2 * i], refs[sc + 6 + 2 * i + 1])
                 for i, w in enumerate(uniq_w)}
        nb = sc + 6 + 2 * len(uniq_w)
        bbufs = {w: refs[nb + i] for i, w in enumerate(uniq_w)}
        accs = {w: refs[nb + len(uniq_w) + i] for i, w in enumerate(uniq_w)}
        nse = nb + 2 * len(uniq_w)
        gsems = refs[nse:nse + 2]
        ssems = refs[nse + 2:nse + 4]
        isems = refs[nse + 4:nse + 7]

        c = lax.axis_index("c")
        s = lax.axis_index("s")
        base = (c * NS + s) * EPW
        for i, w in enumerate(widths):
            acc, bbuf = accs[w], bbufs[w]
            pltpu.sync_copy(zrefs[w].at[pl.ds(0, RPSUB)], bbuf)
            for j in range(RPT // RPSUB):
                pltpu.sync_copy(bbuf, acc.at[pl.ds(s * RPT + j * RPSUB, RPSUB)])
            plsc.subcore_barrier()
            _agg_chunk(tables[i], acc, rowp, colp, base, ribufs, cibufs,
                       rbufs[w], gsems, ssems, isems)
            plsc.subcore_barrier()
            for j in range(RPT // RPSUB):
                rj = pl.ds(s * RPT + j * RPSUB, RPSUB)
                pltpu.sync_copy(acc.at[rj], bbuf)
                pltpu.sync_copy(bbuf, outs[i].at[c, rj])
            plsc.subcore_barrier()

    mesh = plsc.VectorSubcoreMesh(core_axis_name="c", subcore_axis_name="s")
    scratch = [pltpu.VMEM((EB,), jnp.int32) for _ in range(6)]
    for w in uniq_w:
        scratch += [pltpu.VMEM((EB, w), F32), pltpu.VMEM((EB, w), F32)]
    scratch += [pltpu.VMEM((RPSUB, w), F32) for w in uniq_w]
    scratch += [pltpu.VMEM_SHARED((ACC_ROWS, w), F32) for w in uniq_w]
    scratch += [pltpu.SemaphoreType.DMA] * 7
    out_type = tuple(jax.ShapeDtypeStruct((NC, ACC_ROWS, w), F32) for w in widths)

    return pl.kernel(body, out_type=out_type, mesh=mesh,
                     scratch_types=scratch, name=name)


def _tc_a_body(x_ref, degp_ref, GW1_ref, EW1_ref, Eb1_ref,
               t0_ref, t1_ref, t2_ref, t3_ref, dis_ref):
    x = x_ref[...]
    deg = degp_ref[:, 0] + degp_ref[:, 1] + 1.0
    disc = lax.rsqrt(deg)[:, None]
    dis_ref[...] = disc
    g1 = disc * _matmul_t(x, GW1_ref[...])
    t3_ref[...] = jnp.concatenate(
        [g1, jnp.zeros((g1.shape[0], HID - GATE_HID), F32)], axis=1)
    for i, k in enumerate(CURVS):
        xm = _expmap0(x, k)
        xt = _kappa_dense(xm, EW1_ref[i], Eb1_ref[i], k)
        [t0_ref, t1_ref, t2_ref][i][...] = disc * xt


def _tc_a(x, degp, GW1, EW1, Eb1):
    full = lambda a: pl.BlockSpec(a.shape, lambda i: (0,) * a.ndim)
    return pl.pallas_call(
        _tc_a_body,
        grid=(NGRID,),
        in_specs=[
            pl.BlockSpec((RB, IN_DIM), lambda i: (i, 0)),
            pl.BlockSpec((RB, NC), lambda i: (i, 0)),
            full(GW1), full(EW1), full(Eb1),
        ],
        out_specs=[
            pl.BlockSpec((RB, HID), lambda i: (i, 0)),
            pl.BlockSpec((RB, HID), lambda i: (i, 0)),
            pl.BlockSpec((RB, HID), lambda i: (i, 0)),
            pl.BlockSpec((RB, HID), lambda i: (i, 0)),
            pl.BlockSpec((RB, 1), lambda i: (i, 0)),
        ],
        out_shape=[
            jax.ShapeDtypeStruct((N, HID), F32),
            jax.ShapeDtypeStruct((N, HID), F32),
            jax.ShapeDtypeStruct((N, HID), F32),
            jax.ShapeDtypeStruct((N, HID), F32),
            jax.ShapeDtypeStruct((N, 1), F32),
        ],
        name="tc_a",
    )(x, degp, GW1, EW1, Eb1)


def _tc_b_body(a0_ref, a1_ref, a2_ref, a3_ref, t0_ref, t1_ref, t2_ref, t3_ref,
               dis_ref, Gb1_ref, GW2_ref, EW2_ref, Eb2_ref, u0_ref, u1_ref):
    disc = dis_ref[...]
    og = disc * (a3_ref[0, :, :GATE_HID] + a3_ref[1, :, :GATE_HID] +
                 t3_ref[:, :GATE_HID])
    h1 = jax.nn.relu(og + Gb1_ref[...][None, :])
    fg = disc * _matmul_t(h1, GW2_ref[...])
    es = []
    for i, k in enumerate(CURVS):
        a = [a0_ref, a1_ref, a2_ref][i]
        t = [t0_ref, t1_ref, t2_ref][i]
        o = disc * (a[0] + a[1] + t[...])
        xm1 = _expmap0(o, k)
        es.append(disc * _kappa_dense(xm1, EW2_ref[i], Eb2_ref[i], k))
    u0_ref[...] = jnp.concatenate([fg, es[0]], axis=1)
    u1_ref[...] = jnp.concatenate([es[1], es[2]], axis=1)


def _tc_b(a1s, ts, dis, Gb1, GW2, EW2, Eb2):
    full = lambda a: pl.BlockSpec(a.shape, lambda i: (0,) * a.ndim)
    specs = [pl.BlockSpec((NC, RB, HID), lambda i: (0, i, 0))] * 4
    specs += [pl.BlockSpec((RB, HID), lambda i: (i, 0))] * 4
    specs += [pl.BlockSpec((RB, 1), lambda i: (i, 0))]
    specs += [full(Gb1), full(GW2), full(EW2), full(Eb2)]
    return pl.pallas_call(
        _tc_b_body,
        grid=(NGRID,),
        in_specs=specs,
        out_specs=[pl.BlockSpec((RB, 2 * OUT), lambda i: (i, 0))] * 2,
        out_shape=[jax.ShapeDtypeStruct((N, 2 * OUT), F32)] * 2,
        name="tc_b",
    )(*a1s, *ts, dis, Gb1, GW2, EW2, Eb2)


def _tc_c_body(a0_ref, a1_ref, u0_ref, u1_ref, dis_ref, Gb2_ref, batch_ref,
               u_ref, tau_ref, out_ref, zacc, cacc):
    i = pl.program_id(0)
    disc = dis_ref[...]
    o0 = disc * (a0_ref[0] + a0_ref[1] + u0_ref[...])
    o1 = disc * (a1_ref[0] + a1_ref[1] + u1_ref[...])
    h2 = jax.nn.relu(o0[:, :OUT] + Gb2_ref[...][None, :])
    zs = [h2]
    for idx, k in enumerate(CURVS):
        o = [o0[:, OUT:], o1[:, :OUT], o1[:, OUT:]][idx]
        zs.append(_logmap0(_expmap0(o, k), k))
    cat = jnp.concatenate(zs, axis=1)
    b = batch_ref[...][:, 0]
    gid = lax.broadcasted_iota(jnp.int32, (NG, RB), 0)
    P = (b[None, :] == gid).astype(F32)
    zpart = lax.dot_general(P, cat, (((1,), (0,)), ((), ())),
                            preferred_element_type=F32)
    cpart = jnp.sum(P, axis=1)

    @pl.when(i == 0)
    def _():
        zacc[...] = jnp.zeros_like(zacc)
        cacc[...] = jnp.zeros_like(cacc)
        out_ref[...] = jnp.zeros_like(out_ref)

    zacc[...] += zpart
    cacc[...] += cpart

    @pl.when(i == NGRID - 1)
    def _():
        cnt = jnp.clip(cacc[...], 1.0, None)[:, None]
        Z = zacc[...] / cnt
        hg = Z[:, :OUT]
        tau_raw = tau_ref[...]
        tau = jnp.clip(jnp.maximum(tau_raw, 0.0) +
                       jnp.log1p(jnp.exp(-jnp.abs(tau_raw))) + TAU_MIN,
                       TAU_MIN, TAU_MAX)
        ds = []
        for idx, k in enumerate(CURVS):
            zz = _expmap0(hg, k)
            yy = _expmap0(jnp.broadcast_to(u_ref[idx], hg.shape), k)
            ds.append(_mdist(zz, yy, k))
        d = jnp.stack(ds, axis=-1)
        lg = -d / tau[None, :]
        m = jnp.max(lg, axis=1, keepdims=True)
        e = jnp.exp(lg - m)
        w = e / jnp.sum(e, axis=1, keepdims=True)
        res = jnp.zeros((NG, OUT), F32)
        for idx in range(3):
            res = res + w[:, idx:idx + 1] * Z[:, OUT * (idx + 1):OUT * (idx + 2)]
        out_ref[...] = res


def _tc_c(a2s, us, dis, Gb2, batch, u, tau_raw):
    full = lambda a: pl.BlockSpec(a.shape, lambda i: (0,) * a.ndim)
    specs = [pl.BlockSpec((NC, RB, 2 * OUT), lambda i: (0, i, 0))] * 2
    specs += [pl.BlockSpec((RB, 2 * OUT), lambda i: (i, 0))] * 2
    specs += [pl.BlockSpec((RB, 1), lambda i: (i, 0))]
    specs += [full(Gb2)]
    specs += [pl.BlockSpec((RB, 1), lambda i: (i, 0))]
    specs += [full(u), full(tau_raw)]
    return pl.pallas_call(
        _tc_c_body,
        grid=(NGRID,),
        in_specs=specs,
        out_specs=pl.BlockSpec((NG, OUT), lambda i: (0, 0)),
        out_shape=jax.ShapeDtypeStruct((NG, OUT), F32),
        scratch_shapes=[pltpu.VMEM((NG, 4 * OUT), F32), pltpu.VMEM((NG,), F32)],
        name="tc_c",
    )(*a2s, *us, dis, Gb2, batch, u, tau_raw)


def kernel(x, edge_index, batch, EW1, Eb1, EW2, Eb2, GW1, Gb1, GW2, Gb2, u,
           tau_raw):
    x = x.astype(F32)
    pad = EP - E
    prow = jnp.arange(pad, dtype=jnp.int32) % N
    pcol = DUMMY + jnp.arange(pad, dtype=jnp.int32) % (ACC_ROWS - N)
    row_p = jnp.concatenate([edge_index[0], prow])
    col_p = jnp.concatenate([edge_index[1], pcol])
    z1 = jnp.zeros((ACC_ROWS,), F32)
    z128 = jnp.zeros((ACC_ROWS, HID), F32)

    degp = _sc_deg(col_p, z1).reshape(NC, ACC_ROWS)
    t0, t1, t2, t3, dis = _tc_a(x, degp[:, :N].T, GW1, EW1, Eb1)
    pass1 = _make_sc_pass((HID, HID, HID, HID), "sc_pass1")
    a10, a11, a12, a13 = pass1(t0, t1, t2, t3, row_p, col_p, z128)
    u0, u1 = _tc_b([a10, a11, a12, a13], [t0, t1, t2, t3], dis, Gb1, GW2, EW2,
                   Eb2)
    pass2 = _make_sc_pass((2 * OUT, 2 * OUT), "sc_pass2")
    a20, a21 = pass2(u0, u1, row_p, col_p, z128)
    return _tc_c([a20, a21], [u0, u1], dis, Gb2, batch[:, None], u, tau_raw)


# speedup vs baseline: 18.7994x; 1.0075x over previous
import functools
import numpy as np
import jax
import jax.numpy as jnp
from jax import lax
from jax.experimental import pallas as pl
from jax.experimental.pallas import tpu as pltpu
from jax.experimental.pallas import tpu_sc as plsc

N = 10000
E = 320000
IN_DIM = 128
HID = 128
OUT = 64
NG = 64
GATE_HID = 32
CURVS = (-1.0, 0.0, 1.0)
TAU_MIN = 0.05
TAU_MAX = 10.0
EPS = 1e-15

F32 = jnp.float32

NC = 2
NS = 16
NW = NC * NS
EB = 128
NBATCH = -(-E // (EB * NW))
EPW = NBATCH * EB
EP = EPW * NW
ACC_ROWS = 10240
RPT = ACC_ROWS // NS
RPSUB = 80
DUMMY = N

RB = 2000
NGRID = N // RB


def _norm(v):
    return jnp.sqrt(jnp.clip(jnp.sum(v * v, axis=-1, keepdims=True), EPS, None))


def _atan_pos(z):
    t = z
    for _ in range(3):
        t = t / (1.0 + jnp.sqrt(1.0 + t * t))
    t2 = t * t
    p = t * (1.0 + t2 * (-1.0 / 3.0 + t2 * (0.2 + t2 * (-1.0 / 7.0 + t2 / 9.0))))
    return 8.0 * p


def _atanh(z):
    return 0.5 * (jnp.log1p(z) - jnp.log1p(-z))


def _tan_k(t, k):
    if k > 0:
        sk = np.sqrt(k)
        return jnp.tan(sk * t) / sk
    if k < 0:
        sk = np.sqrt(-k)
        return jnp.tanh(sk * t) / sk
    return t


def _artan_k(t, k):
    if k > 0:
        sk = np.sqrt(k)
        return _atan_pos(sk * t) / sk
    if k < 0:
        sk = np.sqrt(-k)
        return _atanh(jnp.clip(sk * t, -1.0 + 1e-7, 1.0 - 1e-7)) / sk
    return t


def _project(v, k):
    if k < 0:
        maxn = (1.0 - 1e-3) / np.sqrt(-k)
        n = _norm(v)
        return jnp.where(n > maxn, v / n * maxn, v)
    return v


def _expmap0(v, k):
    n = _norm(v)
    return _project(_tan_k(n, k) * v / n, k)


def _logmap0(v, k):
    n = _norm(v)
    return _artan_k(n, k) * v / n


def _mobius_add(x, y, k):
    x2 = jnp.sum(x * x, -1, keepdims=True)
    y2 = jnp.sum(y * y, -1, keepdims=True)
    xy = jnp.sum(x * y, -1, keepdims=True)
    num = (1.0 - 2.0 * k * xy - k * y2) * x + (1.0 + k * x2) * y
    den = 1.0 - 2.0 * k * xy + k * k * x2 * y2
    den = jnp.where(jnp.abs(den) < EPS, EPS, den)
    return num / den


def _mdist(x, y, k):
    return 2.0 * _artan_k(jnp.squeeze(_norm(_mobius_add(-x, y, k)), -1), k)


def _matmul_t(a, w):
    return lax.dot_general(a, w, (((1,), (1,)), ((), ())),
                           preferred_element_type=F32)


def _kappa_dense(o, W, b, k):
    lm = _logmap0(o, k)
    y = _matmul_t(lm, W)
    xl = _expmap0(y, k)
    kb = _expmap0(b[None, :], k)
    xl = _project(_mobius_add(xl, kb, k), k)
    return _logmap0(xl, k)


def _agg_chunk(tbl, acc, rowp, colp, base, ribufs, cibufs, rbufs,
               gsems, ssems, isems):
    def idx_load(b, i):
        off = pl.multiple_of(base + b * EB, EB)
        r = pltpu.async_copy(rowp.at[pl.ds(off, EB)], ribufs[i], isems[i])
        c = pltpu.async_copy(colp.at[pl.ds(off, EB)], cibufs[i], isems[i])
        return (r, c)

    gd = [None, None]
    sd = [None, None]
    isd = [None, None, None]
    isd[0] = idx_load(0, 0)
    if NBATCH > 1:
        isd[1] = idx_load(1, 1)
    for d in isd[0]:
        d.wait()
    gd[0] = pltpu.async_copy(tbl.at[ribufs[0]], rbufs[0], gsems[0])
    for b in range(NBATCH):
        p = b & 1
        q = 1 - p
        i0, i1, i2 = b % 3, (b + 1) % 3, (b + 2) % 3
        if b + 1 < NBATCH:
            if sd[q] is not None:
                sd[q].wait()
            for d in isd[i1]:
                d.wait()
            gd[q] = pltpu.async_copy(tbl.at[ribufs[i1]], rbufs[q], gsems[q])
            if b + 2 < NBATCH:
                isd[i2] = idx_load(b + 2, i2)
        gd[p].wait()
        sd[p] = pltpu.async_copy(rbufs[p], acc.at[cibufs[i0]], ssems[p],
                                 add=True)
    for d in sd:
        if d is not None:
            d.wait()


def _deg_body(colp, z1, out, c0, c1, c2, c3, ones_v, bounce, acc, s0, s1,
              i0, i1, i2, i3):
    c = lax.axis_index("c")
    s = lax.axis_index("s")
    base = (c * NS + s) * EPW
    cibufs = (c0, c1, c2, c3)
    isems = (i0, i1, i2, i3)
    ssems = (s0, s1)
    for j in range(EB // 16):
        ones_v[pl.ds(j * 16, 16)] = jnp.ones((16,), F32)
    rows = pl.ds(s * RPT, RPT)
    pltpu.sync_copy(z1.at[rows], bounce)
    pltpu.sync_copy(bounce, acc.at[rows])
    plsc.subcore_barrier()

    def idx_load(b, i):
        off = pl.multiple_of(base + b * EB, EB)
        return pltpu.async_copy(colp.at[pl.ds(off, EB)], cibufs[i], isems[i])

    sd = [None, None]
    isd = [None, None, None, None]
    isd[0] = idx_load(0, 0)
    if NBATCH > 1:
        isd[1] = idx_load(1, 1)
    for b in range(NBATCH):
        p = b & 1
        if sd[p] is not None:
            sd[p].wait()
        isd[b % 4].wait()
        sd[p] = pltpu.async_copy(ones_v, acc.at[cibufs[b % 4]], ssems[p],
                                 add=True)
        if b + 2 < NBATCH:
            isd[(b + 2) % 4] = idx_load(b + 2, (b + 2) % 4)
    for d in sd:
        if d is not None:
            d.wait()
    plsc.subcore_barrier()
    dst = pl.multiple_of(c * ACC_ROWS + s * RPT, 8)
    pltpu.sync_copy(acc.at[rows], bounce)
    pltpu.sync_copy(bounce, out.at[pl.ds(dst, RPT)])


def _sc_deg(col_p, z1):
    mesh = plsc.VectorSubcoreMesh(core_axis_name="c", subcore_axis_name="s")
    return pl.kernel(
        _deg_body,
        out_type=jax.ShapeDtypeStruct((NC * ACC_ROWS,), F32),
        mesh=mesh,
        scratch_types=[pltpu.VMEM((EB,), jnp.int32) for _ in range(4)] + [
            pltpu.VMEM((EB,), F32),
            pltpu.VMEM((RPT,), F32),
            pltpu.VMEM_SHARED((ACC_ROWS,), F32),
        ] + [pltpu.SemaphoreType.DMA] * 6,
        name="sc_deg",
    )(col_p, z1)


def _make_sc_pass(widths, name):
    n_ch = len(widths)
    uniq_w = sorted(set(widths), reverse=True)

    def body(*refs):
        tables = refs[:n_ch]
        rowp, colp = refs[n_ch], refs[n_ch + 1]
        zrefs = {w: refs[n_ch + 2 + i] for i, w in enumerate(uniq_w)}
        outs = refs[n_ch + 2 + len(uniq_w):n_ch + 2 + len(uniq_w) + n_ch]
        sc = n_ch + 2 + len(uniq_w) + n_ch
        ribufs = refs[sc:sc + 3]
        cibufs = refs[sc + 3:sc + 6]
        rbufs = {w: (refs[sc + 6 + 2 * i], refs[sc + 6 + 2 * i + 1])
                 for i, w in enumerate(uniq_w)}
        nb = sc + 6 + 2 * len(uniq_w)
        bbufs = {w: refs[nb + i] for i, w in enumerate(uniq_w)}
        accs = {w: refs[nb + len(uniq_w) + i] for i, w in enumerate(uniq_w)}
        nse = nb + 2 * len(uniq_w)
        gsems = refs[nse:nse + 2]
        ssems = refs[nse + 2:nse + 4]
        isems = refs[nse + 4:nse + 7]

        c = lax.axis_index("c")
        s = lax.axis_index("s")
        base = (c * NS + s) * EPW
        for i, w in enumerate(widths):
            acc, bbuf = accs[w], bbufs[w]
            pltpu.sync_copy(zrefs[w].at[pl.ds(0, RPSUB)], bbuf)
            for j in range(RPT // RPSUB):
                pltpu.sync_copy(bbuf, acc.at[pl.ds(s * RPT + j * RPSUB, RPSUB)])
            plsc.subcore_barrier()
            _agg_chunk(tables[i], acc, rowp, colp, base, ribufs, cibufs,
                       rbufs[w], gsems, ssems, isems)
            plsc.subcore_barrier()
            for j in range(RPT // RPSUB):
                rj = pl.ds(s * RPT + j * RPSUB, RPSUB)
                pltpu.sync_copy(acc.at[rj], bbuf)
                pltpu.sync_copy(bbuf, outs[i].at[c, rj])
            plsc.subcore_barrier()

    mesh = plsc.VectorSubcoreMesh(core_axis_name="c", subcore_axis_name="s")
    scratch = [pltpu.VMEM((EB,), jnp.int32) for _ in range(6)]
    for w in uniq_w:
        scratch += [pltpu.VMEM((EB, w), F32), pltpu.VMEM((EB, w), F32)]
    scratch += [pltpu.VMEM((RPSUB, w), F32) for w in uniq_w]
    scratch += [pltpu.VMEM_SHARED((ACC_ROWS, w), F32) for w in uniq_w]
    scratch += [pltpu.SemaphoreType.DMA] * 7
    out_type = tuple(jax.ShapeDtypeStruct((NC, ACC_ROWS, w), F32) for w in widths)

    return pl.kernel(body, out_type=out_type, mesh=mesh,
                     scratch_types=scratch, name=name)


def _tc_a1_body(x_ref, GW1_ref, EW1_ref, Eb1_ref, r0_ref, r1_ref, r2_ref,
                r3_ref):
    x = x_ref[...]
    g1 = _matmul_t(x, GW1_ref[...])
    r3_ref[...] = jnp.concatenate(
        [g1, jnp.zeros((g1.shape[0], HID - GATE_HID), F32)], axis=1)
    for i, k in enumerate(CURVS):
        xm = _expmap0(x, k)
        [r0_ref, r1_ref, r2_ref][i][...] = _kappa_dense(xm, EW1_ref[i],
                                                        Eb1_ref[i], k)


def _tc_a1(x, GW1, EW1, Eb1):
    full = lambda a: pl.BlockSpec(a.shape, lambda i: (0,) * a.ndim)
    return pl.pallas_call(
        _tc_a1_body,
        grid=(NGRID,),
        in_specs=[
            pl.BlockSpec((RB, IN_DIM), lambda i: (i, 0)),
            full(GW1), full(EW1), full(Eb1),
        ],
        out_specs=[pl.BlockSpec((RB, HID), lambda i: (i, 0))] * 4,
        out_shape=[jax.ShapeDtypeStruct((N, HID), F32)] * 4,
        name="tc_a1",
    )(x, GW1, EW1, Eb1)


def _tc_a2_body(degp_ref, r0_ref, r1_ref, r2_ref, r3_ref,
                t0_ref, t1_ref, t2_ref, t3_ref, dis_ref):
    deg = degp_ref[:, 0] + degp_ref[:, 1] + 1.0
    disc = lax.rsqrt(deg)[:, None]
    dis_ref[...] = disc
    t0_ref[...] = disc * r0_ref[...]
    t1_ref[...] = disc * r1_ref[...]
    t2_ref[...] = disc * r2_ref[...]
    t3_ref[...] = disc * r3_ref[...]


def _tc_a2(degp, rs):
    return pl.pallas_call(
        _tc_a2_body,
        grid=(NGRID,),
        in_specs=[pl.BlockSpec((RB, NC), lambda i: (i, 0))] +
                 [pl.BlockSpec((RB, HID), lambda i: (i, 0))] * 4,
        out_specs=[pl.BlockSpec((RB, HID), lambda i: (i, 0))] * 4 +
                  [pl.BlockSpec((RB, 1), lambda i: (i, 0))],
        out_shape=[jax.ShapeDtypeStruct((N, HID), F32)] * 4 +
                  [jax.ShapeDtypeStruct((N, 1), F32)],
        name="tc_a2",
    )(degp, *rs)


def _tc_b_body(a0_ref, a1_ref, a2_ref, a3_ref, t0_ref, t1_ref, t2_ref, t3_ref,
               dis_ref, Gb1_ref, GW2_ref, EW2_ref, Eb2_ref, u0_ref, u1_ref):
    disc = dis_ref[...]
    og = disc * (a3_ref[0, :, :GATE_HID] + a3_ref[1, :, :GATE_HID] +
                 t3_ref[:, :GATE_HID])
    h1 = jax.nn.relu(og + Gb1_ref[...][None, :])
    fg = disc * _matmul_t(h1, GW2_ref[...])
    es = []
    for i, k in enumerate(CURVS):
        a = [a0_ref, a1_ref, a2_ref][i]
        t = [t0_ref, t1_ref, t2_ref][i]
        o = disc * (a[0] + a[1] + t[...])
        xm1 = _expmap0(o, k)
        es.append(disc * _kappa_dense(xm1, EW2_ref[i], Eb2_ref[i], k))
    u0_ref[...] = jnp.concatenate([fg, es[0]], axis=1)
    u1_ref[...] = jnp.concatenate([es[1], es[2]], axis=1)


def _tc_b(a1s, ts, dis, Gb1, GW2, EW2, Eb2):
    full = lambda a: pl.BlockSpec(a.shape, lambda i: (0,) * a.ndim)
    specs = [pl.BlockSpec((NC, RB, HID), lambda i: (0, i, 0))] * 4
    specs += [pl.BlockSpec((RB, HID), lambda i: (i, 0))] * 4
    specs += [pl.BlockSpec((RB, 1), lambda i: (i, 0))]
    specs += [full(Gb1), full(GW2), full(EW2), full(Eb2)]
    return pl.pallas_call(
        _tc_b_body,
        grid=(NGRID,),
        in_specs=specs,
        out_specs=[pl.BlockSpec((RB, 2 * OUT), lambda i: (i, 0))] * 2,
        out_shape=[jax.ShapeDtypeStruct((N, 2 * OUT), F32)] * 2,
        name="tc_b",
    )(*a1s, *ts, dis, Gb1, GW2, EW2, Eb2)


def _tc_c_body(a0_ref, a1_ref, u0_ref, u1_ref, dis_ref, Gb2_ref, batch_ref,
               u_ref, tau_ref, out_ref, zacc, cacc):
    i = pl.program_id(0)
    disc = dis_ref[...]
    o0 = disc * (a0_ref[0] + a0_ref[1] + u0_ref[...])
    o1 = disc * (a1_ref[0] + a1_ref[1] + u1_ref[...])
    h2 = jax.nn.relu(o0[:, :OUT] + Gb2_ref[...][None, :])
    zs = [h2]
    for idx, k in enumerate(CURVS):
        o = [o0[:, OUT:], o1[:, :OUT], o1[:, OUT:]][idx]
        zs.append(_logmap0(_expmap0(o, k), k))
    cat = jnp.concatenate(zs, axis=1)
    b = batch_ref[...][:, 0]
    gid = lax.broadcasted_iota(jnp.int32, (NG, RB), 0)
    P = (b[None, :] == gid).astype(F32)
    zpart = lax.dot_general(P, cat, (((1,), (0,)), ((), ())),
                            preferred_element_type=F32)
    cpart = jnp.sum(P, axis=1)

    @pl.when(i == 0)
    def _():
        zacc[...] = jnp.zeros_like(zacc)
        cacc[...] = jnp.zeros_like(cacc)
        out_ref[...] = jnp.zeros_like(out_ref)

    zacc[...] += zpart
    cacc[...] += cpart

    @pl.when(i == NGRID - 1)
    def _():
        cnt = jnp.clip(cacc[...], 1.0, None)[:, None]
        Z = zacc[...] / cnt
        hg = Z[:, :OUT]
        tau_raw = tau_ref[...]
        tau = jnp.clip(jnp.maximum(tau_raw, 0.0) +
                       jnp.log1p(jnp.exp(-jnp.abs(tau_raw))) + TAU_MIN,
                       TAU_MIN, TAU_MAX)
        ds = []
        for idx, k in enumerate(CURVS):
            zz = _expmap0(hg, k)
            yy = _expmap0(jnp.broadcast_to(u_ref[idx], hg.shape), k)
            ds.append(_mdist(zz, yy, k))
        d = jnp.stack(ds, axis=-1)
        lg = -d / tau[None, :]
        m = jnp.max(lg, axis=1, keepdims=True)
        e = jnp.exp(lg - m)
        w = e / jnp.sum(e, axis=1, keepdims=True)
        res = jnp.zeros((NG, OUT), F32)
        for idx in range(3):
            res = res + w[:, idx:idx + 1] * Z[:, OUT * (idx + 1):OUT * (idx + 2)]
        out_ref[...] = res


def _tc_c(a2s, us, dis, Gb2, batch, u, tau_raw):
    full = lambda a: pl.BlockSpec(a.shape, lambda i: (0,) * a.ndim)
    specs = [pl.BlockSpec((NC, RB, 2 * OUT), lambda i: (0, i, 0))] * 2
    specs += [pl.BlockSpec((RB, 2 * OUT), lambda i: (i, 0))] * 2
    specs += [pl.BlockSpec((RB, 1), lambda i: (i, 0))]
    specs += [full(Gb2)]
    specs += [pl.BlockSpec((RB, 1), lambda i: (i, 0))]
    specs += [full(u), full(tau_raw)]
    return pl.pallas_call(
        _tc_c_body,
        grid=(NGRID,),
        in_specs=specs,
        out_specs=pl.BlockSpec((NG, OUT), lambda i: (0, 0)),
        out_shape=jax.ShapeDtypeStruct((NG, OUT), F32),
        scratch_shapes=[pltpu.VMEM((NG, 4 * OUT), F32), pltpu.VMEM((NG,), F32)],
        name="tc_c",
    )(*a2s, *us, dis, Gb2, batch, u, tau_raw)


def kernel(x, edge_index, batch, EW1, Eb1, EW2, Eb2, GW1, Gb1, GW2, Gb2, u,
           tau_raw):
    x = x.astype(F32)
    pad = EP - E
    prow = jnp.arange(pad, dtype=jnp.int32) % N
    pcol = DUMMY + jnp.arange(pad, dtype=jnp.int32) % (ACC_ROWS - N)
    row_p = jnp.concatenate([edge_index[0], prow])
    col_p = jnp.concatenate([edge_index[1], pcol])
    z1 = jnp.zeros((ACC_ROWS,), F32)
    z128 = jnp.zeros((ACC_ROWS, HID), F32)

    rs = _tc_a1(x, GW1, EW1, Eb1)
    degp = _sc_deg(col_p, z1).reshape(NC, ACC_ROWS)
    t0, t1, t2, t3, dis = _tc_a2(degp[:, :N].T, rs)
    pass1 = _make_sc_pass((HID, HID, HID, HID), "sc_pass1")
    a10, a11, a12, a13 = pass1(t0, t1, t2, t3, row_p, col_p, z128)
    u0, u1 = _tc_b([a10, a11, a12, a13], [t0, t1, t2, t3], dis, Gb1, GW2, EW2,
                   Eb2)
    pass2 = _make_sc_pass((2 * OUT, 2 * OUT), "sc_pass2")
    a20, a21 = pass2(u0, u1, row_p, col_p, z128)
    return _tc_c([a20, a21], [u0, u1], dis, Gb2, batch[:, None], u, tau_raw)
